# baseline (speedup 1.0000x reference)
import functools

import jax
import jax.numpy as jnp
from jax import lax
from jax.experimental import pallas as pl
from jax.experimental.pallas import tpu as pltpu
from jax.experimental.pallas import tpu_sc as plsc

BATCH = 16384
EMBED_DIM = 64
WPR = EMBED_DIM // 2
ROW = 4 * WPR
NUM_CORES = 2
NUM_SUBCORES = 16
NUM_WORKERS = NUM_CORES * NUM_SUBCORES
BPW = BATCH // NUM_WORKERS
CHUNK = 128
NCHUNK = BPW // CHUNK
LANES = 16
NBUF = 2


def _dot_body(user_hbm, mission_hbm, tab_hbm, out_hbm,
              uidx, midx, upair, mpair, ubuf, mbuf, out_v, sem):
    wid = lax.axis_index("s") * NUM_CORES + lax.axis_index("c")
    base = wid * BPW

    pltpu.sync_copy(user_hbm.at[pl.ds(base, BPW)], uidx)
    pltpu.sync_copy(mission_hbm.at[pl.ds(base, BPW)], midx)

    for c in range(BPW // LANES):
        sl = pl.ds(c * LANES, LANES)
        upair[sl] = lax.shift_right_logical(uidx[sl], 1)
        mpair[sl] = lax.shift_right_logical(midx[sl], 1)

    def fire(c, buf):
        sl = pl.ds(c * CHUNK, CHUNK)
        cp_u = pltpu.async_copy(tab_hbm.at[upair.at[sl]], ubuf.at[buf], sem)
        cp_m = pltpu.async_copy(tab_hbm.at[mpair.at[sl]], mbuf.at[buf], sem)
        return cp_u, cp_m

    def compute(c, buf):
        sl = pl.ds(c * CHUNK, CHUNK)
        one = jnp.full((LANES,), 1, jnp.int32)
        himask = jnp.full((LANES,), jnp.int32(-65536))

        for g in range(CHUNK // LANES):
            gsl = pl.ds(c * CHUNK + g * LANES, LANES)
            rv = jnp.full((LANES,), g * LANES, jnp.int32) + lax.iota(
                jnp.int32, LANES)
            uhalf = lax.bitwise_and(uidx[gsl], one) * (2 * WPR)
            mhalf = lax.bitwise_and(midx[gsl], one) * (2 * WPR) + WPR

            def body(w, acc):
                wv = jnp.full((LANES,), w, jnp.int32)
                uw = plsc.load_gather(ubuf.at[buf], [rv, uhalf + wv])
                mw = plsc.load_gather(mbuf.at[buf], [rv, mhalf + wv])
                ulo = plsc.bitcast(lax.shift_left(uw, 16), jnp.float32)
                uhi = plsc.bitcast(lax.bitwise_and(uw, himask), jnp.float32)
                mlo = plsc.bitcast(lax.shift_left(mw, 16), jnp.float32)
                mhi = plsc.bitcast(lax.bitwise_and(mw, himask), jnp.float32)
                return acc + ulo * mlo + uhi * mhi

            acc = lax.fori_loop(0, WPR, body,
                                jnp.zeros((LANES,), jnp.float32), unroll=8)
            out_v[gsl] = acc

    pending = fire(0, 0)
    for c in range(NCHUNK):
        if c + 1 < NCHUNK:
            nxt = fire(c + 1, (c + 1) % NBUF)
        for cp in pending:
            cp.wait()
        compute(c, c % NBUF)
        if c + 1 < NCHUNK:
            pending = nxt

    pltpu.sync_copy(out_v, out_hbm.at[pl.ds(base, BPW)])


def _pack_words(table16):
    lo = table16[:, 0::2].astype(jnp.uint32)
    hi = table16[:, 1::2].astype(jnp.uint32)
    return (lo | (hi << 16)).astype(jnp.int32)


@functools.partial(jax.jit, static_argnames=())
def kernel(user, mission, user_table, mission_table):
    mesh = plsc.VectorSubcoreMesh(core_axis_name="c", subcore_axis_name="s")
    run = functools.partial(
        pl.kernel,
        mesh=mesh,
        compiler_params=pltpu.CompilerParams(needs_layout_passes=False),
        out_type=jax.ShapeDtypeStruct((BATCH,), jnp.float32),
        scratch_types=[
            pltpu.VMEM((BPW,), jnp.int32),
            pltpu.VMEM((BPW,), jnp.int32),
            pltpu.VMEM((BPW,), jnp.int32),
            pltpu.VMEM((BPW,), jnp.int32),
            pltpu.VMEM((NBUF, CHUNK, ROW), jnp.int32),
            pltpu.VMEM((NBUF, CHUNK, ROW), jnp.int32),
            pltpu.VMEM((BPW,), jnp.float32),
            pltpu.SemaphoreType.DMA,
        ],
    )(_dot_body)
    u16 = jax.lax.bitcast_convert_type(
        user_table.astype(jnp.bfloat16), jnp.uint16)
    m16 = jax.lax.bitcast_convert_type(
        mission_table.astype(jnp.bfloat16), jnp.uint16)
    words = jnp.concatenate([_pack_words(u16), _pack_words(m16)], axis=1)
    packed = words.reshape(-1, ROW)
    return run(user, mission, packed)


# speedup vs baseline: 3.2932x; 3.2932x over previous
import functools

import jax
import jax.numpy as jnp
from jax import lax
from jax.experimental import pallas as pl
from jax.experimental.pallas import tpu as pltpu
from jax.experimental.pallas import tpu_sc as plsc

BATCH = 16384
EMBED_DIM = 64
ROW = 2 * EMBED_DIM
NUM_CORES = 2
NUM_SUBCORES = 16
NUM_WORKERS = NUM_CORES * NUM_SUBCORES
BPW = BATCH // NUM_WORKERS
CHUNK = 128
NCHUNK = BPW // CHUNK
LANES = 16
NBUF = 2

_CP = pltpu.CompilerParams(needs_layout_passes=False)
_MESH = dict(core_axis_name="c", subcore_axis_name="s")


def _gather_body(idx_hbm, tab_hbm, out_hbm, idx_v, buf, sem):
    wid = lax.axis_index("s") * NUM_CORES + lax.axis_index("c")
    base = wid * BPW
    pltpu.sync_copy(idx_hbm.at[pl.ds(base, BPW)], idx_v)

    def fire(c, b):
        sl = pl.ds(c * CHUNK, CHUNK)
        return pltpu.async_copy(tab_hbm.at[idx_v.at[sl]], buf.at[b], sem)

    pending = fire(0, 0)
    for c in range(NCHUNK):
        if c + 1 < NCHUNK:
            nxt = fire(c + 1, (c + 1) % NBUF)
        pending.wait()
        pltpu.sync_copy(buf.at[c % NBUF],
                        out_hbm.at[pl.ds(base + c * CHUNK, CHUNK)])
        if c + 1 < NCHUNK:
            pending = nxt


def _dot_body(u_hbm, m_hbm, out_hbm, ubuf, mbuf, out_v, sem):
    wid = lax.axis_index("s") * NUM_CORES + lax.axis_index("c")
    base = wid * BPW
    for c in range(NCHUNK):
        pltpu.sync_copy(u_hbm.at[pl.ds(base + c * CHUNK, CHUNK)], ubuf)
        pltpu.sync_copy(m_hbm.at[pl.ds(base + c * CHUNK, CHUNK)], mbuf)
        for g in range(CHUNK // LANES):
            rv = jnp.full((LANES,), g * LANES, jnp.int32) + lax.iota(
                jnp.int32, LANES)

            def body(d, acc):
                dv = jnp.full((LANES,), d, jnp.int32)
                u = plsc.load_gather(ubuf, [rv, dv])
                m = plsc.load_gather(mbuf, [rv, dv])
                return acc + u * m

            acc = lax.fori_loop(0, EMBED_DIM, body,
                                jnp.zeros((LANES,), jnp.float32), unroll=8)
            out_v[pl.ds(c * CHUNK + g * LANES, LANES)] = acc
    pltpu.sync_copy(out_v, out_hbm.at[pl.ds(base, BPW)])


def _make_gather():
    mesh = plsc.VectorSubcoreMesh(**_MESH)
    return functools.partial(
        pl.kernel, mesh=mesh, compiler_params=_CP,
        out_type=jax.ShapeDtypeStruct((BATCH, ROW), jnp.float32),
        scratch_types=[
            pltpu.VMEM((BPW,), jnp.int32),
            pltpu.VMEM((NBUF, CHUNK, ROW), jnp.float32),
            pltpu.SemaphoreType.DMA,
        ],
    )(_gather_body)


def _make_dot():
    mesh = plsc.VectorSubcoreMesh(**_MESH)
    return functools.partial(
        pl.kernel, mesh=mesh, compiler_params=_CP,
        out_type=jax.ShapeDtypeStruct((BATCH,), jnp.float32),
        scratch_types=[
            pltpu.VMEM((CHUNK, ROW), jnp.float32),
            pltpu.VMEM((CHUNK, ROW), jnp.float32),
            pltpu.VMEM((BPW,), jnp.float32),
            pltpu.SemaphoreType.DMA,
        ],
    )(_dot_body)


@functools.partial(jax.jit, static_argnames=())
def kernel(user, mission, user_table, mission_table):
    upad = jnp.pad(user_table, ((0, 0), (0, EMBED_DIM)))
    mpad = jnp.pad(mission_table, ((0, 0), (0, EMBED_DIM)))
    u_emb = _make_gather()(user, upad)
    m_emb = _make_gather()(mission, mpad)
    return _make_dot()(u_emb, m_emb)


# speedup vs baseline: 3.7957x; 1.1526x over previous
import functools

import jax
import jax.numpy as jnp
from jax import lax
from jax.experimental import pallas as pl
from jax.experimental.pallas import tpu as pltpu
from jax.experimental.pallas import tpu_sc as plsc

BATCH = 16384
EMBED_DIM = 64
ROW = 2 * EMBED_DIM
NUM_CORES = 2
NUM_SUBCORES = 16
NUM_WORKERS = NUM_CORES * NUM_SUBCORES
BPW = BATCH // NUM_WORKERS
CHUNK = 128
NCHUNK = BPW // CHUNK
LANES = 16
NBUF = 2


def _dot_body(user_hbm, mission_hbm, tab_hbm, out_hbm,
              uidx, midx, ubuf, mbuf, out_v, sem):
    wid = lax.axis_index("s") * NUM_CORES + lax.axis_index("c")
    base = wid * BPW

    pltpu.sync_copy(user_hbm.at[pl.ds(base, BPW)], uidx)
    pltpu.sync_copy(mission_hbm.at[pl.ds(base, BPW)], midx)

    def fire(c, buf):
        sl = pl.ds(c * CHUNK, CHUNK)
        cp_u = pltpu.async_copy(tab_hbm.at[uidx.at[sl]], ubuf.at[buf], sem)
        cp_m = pltpu.async_copy(tab_hbm.at[midx.at[sl]], mbuf.at[buf], sem)
        return cp_u, cp_m

    def compute(c, buf):
        for g in range(CHUNK // LANES):
            rv = jnp.full((LANES,), g * LANES, jnp.int32) + lax.iota(
                jnp.int32, LANES)

            def body(d, acc):
                dv = jnp.full((LANES,), d, jnp.int32)
                u = plsc.load_gather(ubuf.at[buf], [rv, dv])
                m = plsc.load_gather(mbuf.at[buf], [rv, dv + EMBED_DIM])
                return acc + u * m

            acc = lax.fori_loop(0, EMBED_DIM, body,
                                jnp.zeros((LANES,), jnp.float32), unroll=8)
            out_v[pl.ds(c * CHUNK + g * LANES, LANES)] = acc

    pending = fire(0, 0)
    for c in range(NCHUNK):
        if c + 1 < NCHUNK:
            nxt = fire(c + 1, (c + 1) % NBUF)
        for cp in pending:
            cp.wait()
        compute(c, c % NBUF)
        if c + 1 < NCHUNK:
            pending = nxt

    pltpu.sync_copy(out_v, out_hbm.at[pl.ds(base, BPW)])


@functools.partial(jax.jit, static_argnames=())
def kernel(user, mission, user_table, mission_table):
    mesh = plsc.VectorSubcoreMesh(core_axis_name="c", subcore_axis_name="s")
    run = functools.partial(
        pl.kernel,
        mesh=mesh,
        compiler_params=pltpu.CompilerParams(needs_layout_passes=False),
        out_type=jax.ShapeDtypeStruct((BATCH,), jnp.float32),
        scratch_types=[
            pltpu.VMEM((BPW,), jnp.int32),
            pltpu.VMEM((BPW,), jnp.int32),
            pltpu.VMEM((NBUF, CHUNK, ROW), jnp.float32),
            pltpu.VMEM((NBUF, CHUNK, ROW), jnp.float32),
            pltpu.VMEM((BPW,), jnp.float32),
            pltpu.SemaphoreType.DMA,
        ],
    )(_dot_body)
    big = jnp.concatenate([user_table.T, mission_table.T], axis=0).T
    return run(user, mission, big)


# speedup vs baseline: 3.7982x; 1.0007x over previous
import functools

import jax
import jax.numpy as jnp
from jax import lax
from jax.experimental import pallas as pl
from jax.experimental.pallas import tpu as pltpu
from jax.experimental.pallas import tpu_sc as plsc

BATCH = 16384
EMBED_DIM = 64
ROW = 2 * EMBED_DIM
NUM_CORES = 2
NUM_SUBCORES = 16
NUM_WORKERS = NUM_CORES * NUM_SUBCORES
BPW = BATCH // NUM_WORKERS
CHUNK = 128
NCHUNK = BPW // CHUNK
LANES = 16
NBUF = 2


def _dot_body(user_hbm, mission_hbm, tab_hbm, out_hbm,
              uidx, midx, ubuf, mbuf, out_v, sem):
    wid = lax.axis_index("s") * NUM_CORES + lax.axis_index("c")
    base = wid * BPW

    pltpu.sync_copy(user_hbm.at[pl.ds(base, BPW)], uidx)
    pltpu.sync_copy(mission_hbm.at[pl.ds(base, BPW)], midx)

    def fire(c, buf):
        sl = pl.ds(c * CHUNK, CHUNK)
        cp_u = pltpu.async_copy(tab_hbm.at[uidx.at[sl]], ubuf.at[buf], sem)
        cp_m = pltpu.async_copy(tab_hbm.at[midx.at[sl]], mbuf.at[buf], sem)
        return cp_u, cp_m

    def compute(c, buf):
        for g in range(CHUNK // LANES):
            rv = jnp.full((LANES,), g * LANES, jnp.int32) + lax.iota(
                jnp.int32, LANES)

            def body(d, acc):
                dv = jnp.full((LANES,), d, jnp.int32)
                u = plsc.load_gather(ubuf.at[buf], [rv, dv])
                m = plsc.load_gather(mbuf.at[buf], [rv, dv + EMBED_DIM])
                return acc + u * m

            acc = lax.fori_loop(0, EMBED_DIM, body,
                                jnp.zeros((LANES,), jnp.float32), unroll=8)
            out_v[pl.ds(c * CHUNK + g * LANES, LANES)] = acc

    pending = fire(0, 0)
    for c in range(NCHUNK):
        if c + 1 < NCHUNK:
            nxt = fire(c + 1, (c + 1) % NBUF)
        for cp in pending:
            cp.wait()
        compute(c, c % NBUF)
        if c + 1 < NCHUNK:
            pending = nxt

    pltpu.sync_copy(out_v, out_hbm.at[pl.ds(base, BPW)])


@functools.partial(jax.jit, static_argnames=())
def kernel(user, mission, user_table, mission_table):
    mesh = plsc.VectorSubcoreMesh(core_axis_name="c", subcore_axis_name="s")
    run = functools.partial(
        pl.kernel,
        mesh=mesh,
        compiler_params=pltpu.CompilerParams(needs_layout_passes=False),
        out_type=jax.ShapeDtypeStruct((BATCH,), jnp.float32),
        scratch_types=[
            pltpu.VMEM((BPW,), jnp.int32),
            pltpu.VMEM((BPW,), jnp.int32),
            pltpu.VMEM((NBUF, CHUNK, ROW), jnp.float32),
            pltpu.VMEM((NBUF, CHUNK, ROW), jnp.float32),
            pltpu.VMEM((BPW,), jnp.float32),
            pltpu.SemaphoreType.DMA,
        ],
    )(_dot_body)
    big = jnp.concatenate([user_table, mission_table], axis=1)
    return run(user, mission, big)
